# Initial kernel scaffold; baseline (speedup 1.0000x reference)
import jax
import jax.numpy as jnp
from jax.experimental import pallas as pl


def kernel(x, edge_index, edge_weight, W1, b1, W2, b2, W3, b3, Wl, bl):
    raise NotImplementedError("write your pallas kernel here")


# speedup vs baseline: 9.9852x; 9.9852x over previous
import functools

import jax
import jax.numpy as jnp
from jax import lax
from jax.experimental import pallas as pl
from jax.experimental.pallas import tpu as pltpu
from jax.experimental.pallas import tpu_sc as plsc

NC = 2
NS = 16
LANES = 16
NWORK = NC * NS
K = 128


def _cdiv(a, b):
    return (a + b - 1) // b


def _bcast_lane(vec16, t):
    dn = lax.GatherDimensionNumbers(offset_dims=(), collapsed_slice_dims=(0,),
                                    start_index_map=(0,))
    idx = jnp.full((LANES,), t, jnp.int32)[:, None]
    return lax.gather(vec16, idx, dn, (1,),
                      mode=lax.GatherScatterMode.PROMISE_IN_BOUNDS)


def _pieces(total, maxpiece):
    out = []
    left = total
    while left > 0:
        p = min(left, maxpiece)
        out.append(p)
        left -= p
    return out


@functools.lru_cache(maxsize=None)
def _make_deg_kernel(nch, n_pad):
    npt = n_pad // NS
    mesh = plsc.VectorSubcoreMesh(core_axis_name="c", subcore_axis_name="s",
                                  num_cores=NC, num_subcores=NS)

    def body(dst_hbm, ew_hbm, out_hbm, dst_v, ew_v, zbuf, deg_sh):
        c = lax.axis_index("c")
        s = lax.axis_index("s")
        wid = c * NS + s
        pltpu.sync_copy(dst_hbm.at[wid], dst_v)
        pltpu.sync_copy(ew_hbm.at[wid], ew_v)

        def zr(i, carry):
            zbuf[pl.ds(i * LANES, LANES)] = jnp.zeros((LANES,), jnp.float32)
            return carry

        lax.fori_loop(0, npt // LANES, zr, 0)
        pltpu.sync_copy(zbuf, deg_sh.at[pl.ds(s * npt, npt)])
        plsc.subcore_barrier()

        def chunk(j, carry):
            pltpu.sync_copy(ew_v.at[j], deg_sh.at[dst_v.at[j]], add=True)
            return carry

        lax.fori_loop(0, nch, chunk, 0)
        plsc.subcore_barrier()
        pltpu.sync_copy(deg_sh.at[pl.ds(s * npt, npt)],
                        out_hbm.at[c, pl.ds(s * npt, npt)])

    return pl.kernel(
        body,
        out_type=jax.ShapeDtypeStruct((NC, n_pad), jnp.float32),
        mesh=mesh,
        scratch_types=[
            pltpu.VMEM((nch, K), jnp.int32),
            pltpu.VMEM((nch, K), jnp.float32),
            pltpu.VMEM((npt,), jnp.float32),
            pltpu.VMEM_SHARED((n_pad,), jnp.float32),
        ],
    )


@functools.lru_cache(maxsize=None)
def _make_prop_kernel(n_pad, h, nch):
    rpt = n_pad // NS
    mesh = plsc.VectorSubcoreMesh(core_axis_name="c", subcore_axis_name="s",
                                  num_cores=NC, num_subcores=NS)

    def body(y_hbm, src_hbm, dst_hbm, ew_hbm, out_hbm,
             src_v, dst_v, ew_f, rows_v, acc_sh, sem):
        c = lax.axis_index("c")
        s = lax.axis_index("s")
        wid = c * NS + s
        pltpu.sync_copy(src_hbm.at[wid], src_v)
        pltpu.sync_copy(dst_hbm.at[wid], dst_v)
        pltpu.sync_copy(ew_hbm.at[wid], ew_f)

        def zrow(r, carry):
            for t in range(h // LANES):
                rows_v[r, pl.ds(t * LANES, LANES)] = jnp.zeros((LANES,), jnp.float32)
            return carry

        lax.fori_loop(0, K, zrow, 0)
        base = s * rpt
        off = 0
        for piece in _pieces(rpt, K):
            pltpu.sync_copy(rows_v.at[pl.ds(0, piece)],
                            acc_sh.at[pl.ds(base + off, piece)])
            off += piece
        plsc.subcore_barrier()

        def chunk(j, carry):
            pltpu.async_copy(y_hbm.at[src_v.at[j]], rows_v, sem).wait()

            def scale(g, c2):
                ew16 = ew_f[pl.ds(j * K + g * LANES, LANES)]
                for t in range(LANES):
                    ewb = _bcast_lane(ew16, t)
                    r = g * LANES + t
                    for q in range(h // LANES):
                        sl = pl.ds(q * LANES, LANES)
                        rows_v[r, sl] = rows_v[r, sl] * ewb
                return c2

            lax.fori_loop(0, K // LANES, scale, 0)
            pltpu.sync_copy(rows_v, acc_sh.at[dst_v.at[j]], add=True)
            return carry

        lax.fori_loop(0, nch, chunk, 0)
        plsc.subcore_barrier()
        off = 0
        for piece in _pieces(rpt, K):
            pltpu.sync_copy(acc_sh.at[pl.ds(base + off, piece)],
                            out_hbm.at[c, pl.ds(base + off, piece)])
            off += piece

    return pl.kernel(
        body,
        out_type=jax.ShapeDtypeStruct((NC, n_pad, h), jnp.float32),
        mesh=mesh,
        scratch_types=[
            pltpu.VMEM((nch, K), jnp.int32),
            pltpu.VMEM((nch, K), jnp.int32),
            pltpu.VMEM((nch * K,), jnp.float32),
            pltpu.VMEM((K, h), jnp.float32),
            pltpu.VMEM_SHARED((n_pad, h), jnp.float32),
            pltpu.SemaphoreType.DMA,
        ],
    )


def _row_spec(bn, w):
    return pl.BlockSpec((bn, w), lambda i: (i, 0))


def _const_spec(shape):
    return pl.BlockSpec(shape, lambda i: tuple(0 for _ in shape))


@functools.lru_cache(maxsize=None)
def _make_tc0(n, d, h, bn):
    def body(dp0_ref, dp1_ref, x_ref, w_ref, dinv_ref, y_ref):
        deg = jnp.maximum(dp0_ref[...] + dp1_ref[...] + 1.0, 1e-12)
        dinv = lax.rsqrt(deg)
        dinv_ref[...] = dinv
        xw = jnp.dot(x_ref[...], w_ref[...], preferred_element_type=jnp.float32)
        y_ref[...] = dinv * xw

    return pl.pallas_call(
        body,
        grid=(n // bn,),
        in_specs=[_row_spec(bn, 1), _row_spec(bn, 1), _row_spec(bn, d),
                  _const_spec((d, h))],
        out_specs=[_row_spec(bn, 1), _row_spec(bn, h)],
        out_shape=[jax.ShapeDtypeStruct((n, 1), jnp.float32),
                   jax.ShapeDtypeStruct((n, h), jnp.float32)],
    )


@functools.lru_cache(maxsize=None)
def _make_tc_mid(n, h, bn, with_res):
    def body(*refs):
        if with_res:
            (a0_ref, a1_ref, y_ref, dinv_ref, b_ref, res_ref, w_ref,
             h_ref, ynext_ref) = refs
        else:
            (a0_ref, a1_ref, y_ref, dinv_ref, b_ref, w_ref,
             h_ref, ynext_ref) = refs
        dinv = dinv_ref[...]
        spre = dinv * (a0_ref[...] + a1_ref[...] + y_ref[...]) + b_ref[...]
        hv = jnp.where(spre > 0, spre, jnp.exp(spre) - 1.0)
        if with_res:
            hv = hv + res_ref[...]
        h_ref[...] = hv
        ynext_ref[...] = dinv * jnp.dot(hv, w_ref[...],
                                        preferred_element_type=jnp.float32)

    in_specs = [_row_spec(bn, h), _row_spec(bn, h), _row_spec(bn, h),
                _row_spec(bn, 1), _const_spec((1, h))]
    if with_res:
        in_specs.append(_row_spec(bn, h))
    in_specs.append(_const_spec((h, h)))
    return pl.pallas_call(
        body,
        grid=(n // bn,),
        in_specs=in_specs,
        out_specs=[_row_spec(bn, h), _row_spec(bn, h)],
        out_shape=[jax.ShapeDtypeStruct((n, h), jnp.float32),
                   jax.ShapeDtypeStruct((n, h), jnp.float32)],
    )


@functools.lru_cache(maxsize=None)
def _make_tc_fin(n, h, bn):
    def body(a0_ref, a1_ref, y_ref, dinv_ref, b_ref, res_ref, wl_ref, bl_ref,
             out_ref):
        dinv = dinv_ref[...]
        spre = dinv * (a0_ref[...] + a1_ref[...] + y_ref[...]) + b_ref[...]
        hv = jnp.where(spre > 0, spre, jnp.exp(spre) - 1.0) + res_ref[...]
        z = jnp.dot(hv, wl_ref[...], preferred_element_type=jnp.float32)
        z = z + bl_ref[...]
        out_ref[...] = 1.0 / (1.0 + jnp.exp(-z))

    return pl.pallas_call(
        body,
        grid=(n // bn,),
        in_specs=[_row_spec(bn, h), _row_spec(bn, h), _row_spec(bn, h),
                  _row_spec(bn, 1), _const_spec((1, h)), _row_spec(bn, h),
                  _const_spec((h, 1)), _const_spec((1, 1))],
        out_specs=_row_spec(bn, 1),
        out_shape=jax.ShapeDtypeStruct((n, 1), jnp.float32),
    )


def kernel(x, edge_index, edge_weight, W1, b1, W2, b2, W3, b3, Wl, bl):
    n, d = x.shape
    h = W1.shape[1]
    e = edge_weight.shape[0]
    nch = _cdiv(e, NWORK * K)
    e_pad = NWORK * nch * K
    n_pad = NS * (_cdiv(_cdiv(n, NS), LANES) * LANES)
    bn = 1000 if n % 1000 == 0 else (n // NS)

    src = edge_index[0]
    dst = edge_index[1]
    pad = e_pad - e
    src3 = jnp.pad(src, (0, pad)).reshape(NWORK, nch, K)
    dst3 = jnp.pad(dst, (0, pad)).reshape(NWORK, nch, K)
    ew3 = jnp.pad(edge_weight, (0, pad)).reshape(NWORK, nch, K)

    degp = _make_deg_kernel(nch, n_pad)(dst3, ew3)
    dp0 = degp[0, :n].reshape(n, 1)
    dp1 = degp[1, :n].reshape(n, 1)

    dinv, y1 = _make_tc0(n, d, h, bn)(dp0, dp1, x, W1)

    ew2 = ew3.reshape(NWORK, nch * K)
    prop = _make_prop_kernel(n_pad, h, nch)
    acc1 = prop(y1, src3, dst3, ew2)
    h1, y2 = _make_tc_mid(n, h, bn, False)(
        acc1[0, :n], acc1[1, :n], y1, dinv, b1.reshape(1, h), W2)
    acc2 = prop(y2, src3, dst3, ew2)
    h2, y3 = _make_tc_mid(n, h, bn, True)(
        acc2[0, :n], acc2[1, :n], y2, dinv, b2.reshape(1, h), h1, W3)
    acc3 = prop(y3, src3, dst3, ew2)
    out = _make_tc_fin(n, h, bn)(
        acc3[0, :n], acc3[1, :n], y3, dinv, b3.reshape(1, h), h2, Wl,
        bl.reshape(1, 1))
    return out
